# baseline (speedup 1.0000x reference)
import functools

import jax
import jax.numpy as jnp
from jax import lax
from jax.experimental import pallas as pl
from jax.experimental.pallas import tpu as pltpu
from jax.experimental.pallas import tpu_sc as plsc

N = 10000
E = 320000
D = 128
NC = 2
NS = 16
NW = NC * NS
EPW = E // NW
BLK = 80
NBLK = EPW // BLK
GC = 5
IC = NBLK // GC
R0 = 632
NH = 10240
RLAST = N - (NS - 1) * R0


def _sc_aggregate(with_deg):
    mesh = plsc.VectorSubcoreMesh(core_axis_name="c", subcore_axis_name="s")

    out_type = [jax.ShapeDtypeStruct((NC, N, D), jnp.float32)]
    scratch = [
        pltpu.VMEM((IC * BLK,), jnp.int32),
        pltpu.VMEM((IC * BLK,), jnp.int32),
        pltpu.VMEM((BLK, D), jnp.float32),
        pltpu.VMEM((BLK, D), jnp.float32),
        pltpu.VMEM((BLK, D), jnp.float32),
        pltpu.VMEM_SHARED((N, D), jnp.float32),
        pltpu.SemaphoreType.DMA,
        pltpu.SemaphoreType.DMA,
        pltpu.SemaphoreType.DMA,
        pltpu.SemaphoreType.DMA,
        pltpu.SemaphoreType.DMA,
        pltpu.SemaphoreType.DMA,
    ]
    if with_deg:
        out_type.append(jax.ShapeDtypeStruct((NW, NH), jnp.float32))
        scratch += [
            pltpu.VMEM((NH,), jnp.float32),
        ]

    @functools.partial(
        pl.kernel, out_type=out_type, mesh=mesh, scratch_types=scratch,
        compiler_params=pltpu.CompilerParams(needs_layout_passes=False))
    def agg(x_hbm, srcf_hbm, dstf_hbm, *rest):
        if with_deg:
            (out_hbm, deg_hbm, src_v, dst_v, rows0, rows1, rows2,
             acc_sh, gs0, gs1, gs2, ss0, ss1, ss2, hist_v) = rest
        else:
            (out_hbm, src_v, dst_v, rows0, rows1, rows2,
             acc_sh, gs0, gs1, gs2, ss0, ss1, ss2) = rest
        c = lax.axis_index("c")
        s = lax.axis_index("s")
        wid = c * NS + s
        zrows16 = jnp.zeros((16,), jnp.float32)

        def zrow_body(r, carry):
            for j in range(D // 16):
                rows0[r, pl.ds(j * 16, 16)] = zrows16
            return carry

        lax.fori_loop(0, BLK, zrow_body, 0)

        @pl.when(s < NS - 1)
        def _():
            for k in range(R0 // BLK):
                pltpu.sync_copy(rows0, acc_sh.at[pl.ds(s * R0 + k * BLK, BLK)])
            rem = R0 - (R0 // BLK) * BLK
            pltpu.sync_copy(rows0.at[pl.ds(0, rem)],
                            acc_sh.at[pl.ds(s * R0 + R0 - rem, rem)])

        @pl.when(s == NS - 1)
        def _():
            base = (NS - 1) * R0
            for k in range(RLAST // BLK):
                pltpu.sync_copy(rows0, acc_sh.at[pl.ds(base + k * BLK, BLK)])
            rem = RLAST - (RLAST // BLK) * BLK
            pltpu.sync_copy(rows0.at[pl.ds(0, rem)],
                            acc_sh.at[pl.ds(base + RLAST - rem, rem)])

        if with_deg:
            zeros16 = jnp.zeros((16,), jnp.float32)

            def zero_body(i, carry):
                hist_v[pl.ds(i * 16, 16)] = zeros16
                return carry

            lax.fori_loop(0, NH // 16, zero_body, 0)
            ones16 = jnp.full((16,), 1.0, jnp.float32)
        plsc.subcore_barrier()

        def deg_count(i):
            if with_deg:
                for j in range(BLK // 16):
                    chunk = dst_v[pl.ds(i * BLK + j * 16, 16)]
                    plsc.addupdate_scatter(hist_v, [chunk], ones16)

        rows = [rows0, rows1, rows2]
        gs = [gs0, gs1, gs2]
        ss = [ss0, ss1, ss2]

        def issue_gather(j, u):
            pltpu.async_copy(x_hbm.at[src_v.at[pl.ds(j * BLK, BLK)]],
                             rows[u], gs[u])

        def wait_gather(j, u):
            pltpu.make_async_copy(x_hbm.at[src_v.at[pl.ds(j * BLK, BLK)]],
                                  rows[u], gs[u]).wait()

        def issue_scatter(j, u):
            pltpu.async_copy(rows[u], acc_sh.at[dst_v.at[pl.ds(j * BLK, BLK)]],
                             ss[u], add=True)

        def wait_scatter(j, u):
            pltpu.make_async_copy(rows[u], acc_sh.at[dst_v.at[pl.ds(j * BLK, BLK)]],
                                  ss[u]).wait()

        for g in range(GC):
            base = wid * EPW + g * (IC * BLK)
            pltpu.sync_copy(srcf_hbm.at[pl.ds(base, IC * BLK)], src_v)
            pltpu.sync_copy(dstf_hbm.at[pl.ds(base, IC * BLK)], dst_v)
            issue_gather(0, 0)
            issue_gather(1, 1)
            wait_gather(0, 0)
            issue_scatter(0, 0)
            deg_count(0)
            issue_gather(2, 2)
            wait_gather(1, 1)
            issue_scatter(1, 1)
            deg_count(1)
            wait_scatter(0, 0)
            issue_gather(3, 0)
            wait_gather(2, 2)
            issue_scatter(2, 2)
            deg_count(2)

            def body(k, carry):
                for u in range(3):
                    j = 3 * k + u
                    wait_scatter(j - 2, (u + 1) % 3)
                    issue_gather(j + 1, (u + 1) % 3)
                    wait_gather(j, u)
                    issue_scatter(j, u)
                    deg_count(j)
                return carry

            lax.fori_loop(1, IC // 3, body, 0)
            wait_scatter(22, 1)
            wait_gather(24, 0)
            issue_scatter(24, 0)
            deg_count(24)
            wait_scatter(23, 2)
            wait_scatter(24, 0)

        if with_deg:
            pltpu.sync_copy(hist_v, deg_hbm.at[wid])

        plsc.subcore_barrier()

        @pl.when(s < NS - 1)
        def _():
            pltpu.sync_copy(acc_sh.at[pl.ds(s * R0, R0)],
                            out_hbm.at[c, pl.ds(s * R0, R0)])

        @pl.when(s == NS - 1)
        def _():
            pltpu.sync_copy(acc_sh.at[pl.ds((NS - 1) * R0, RLAST)],
                            out_hbm.at[c, pl.ds((NS - 1) * R0, RLAST)])

    return agg


_AGG_DEG = _sc_aggregate(True)
_AGG = _sc_aggregate(False)

_R = 2000


def _tc_degsum(degr):
    def body(deg_ref, o_ref):
        dsum = jnp.sum(deg_ref[...], axis=0)
        o_ref[...] = 1.0 / jnp.maximum(dsum, 1.0)

    return pl.pallas_call(
        body,
        out_shape=jax.ShapeDtypeStruct((NH // 128, 128), jnp.float32),
    )(degr)


def _tc_layer1(aggp, dinv, x, w_l_t, w_r_t, b):
    def body(agg_ref, dinv_ref, x_ref, wl_ref, wr_ref, b_ref, h_ref):
        a = agg_ref[0] + agg_ref[1]
        mean = a * dinv_ref[...]
        h = (jnp.dot(mean, wl_ref[...], preferred_element_type=jnp.float32)
             + jnp.dot(x_ref[...], wr_ref[...], preferred_element_type=jnp.float32)
             + b_ref[...])
        h_ref[...] = jnp.maximum(h, 0.0)

    return pl.pallas_call(
        body,
        grid=(N // _R,),
        in_specs=[
            pl.BlockSpec((NC, _R, D), lambda i: (0, i, 0)),
            pl.BlockSpec((_R, 1), lambda i: (i, 0)),
            pl.BlockSpec((_R, D), lambda i: (i, 0)),
            pl.BlockSpec((D, D), lambda i: (0, 0)),
            pl.BlockSpec((D, D), lambda i: (0, 0)),
            pl.BlockSpec((1, D), lambda i: (0, 0)),
        ],
        out_specs=pl.BlockSpec((_R, D), lambda i: (i, 0)),
        out_shape=jax.ShapeDtypeStruct((N, D), jnp.float32),
    )(aggp, dinv, x, w_l_t, w_r_t, b)


def _tc_layer2(aggp, dinv, h, w_l_t, w_r_t, b):
    def body(agg_ref, dinv_ref, h_ref, wl_ref, wr_ref, b_ref, o_ref):
        a = agg_ref[0] + agg_ref[1]
        mean = a * dinv_ref[...]
        o_ref[...] = (jnp.dot(mean, wl_ref[...], preferred_element_type=jnp.float32)
                      + jnp.dot(h_ref[...], wr_ref[...], preferred_element_type=jnp.float32)
                      + b_ref[...])

    return pl.pallas_call(
        body,
        grid=(N // _R,),
        in_specs=[
            pl.BlockSpec((NC, _R, D), lambda i: (0, i, 0)),
            pl.BlockSpec((_R, 1), lambda i: (i, 0)),
            pl.BlockSpec((_R, D), lambda i: (i, 0)),
            pl.BlockSpec((D, D), lambda i: (0, 0)),
            pl.BlockSpec((D, D), lambda i: (0, 0)),
            pl.BlockSpec((1, D), lambda i: (0, 0)),
        ],
        out_specs=pl.BlockSpec((_R, D), lambda i: (i, 0)),
        out_shape=jax.ShapeDtypeStruct((N, D), jnp.float32),
    )(aggp, dinv, h, w_l_t, w_r_t, b)


def kernel(x, edge_index, W1_l, b1_l, W1_r, W2_l, b2_l, W2_r):
    srcf = edge_index[0]
    dstf = edge_index[1]

    agg1, deg = _AGG_DEG(x, srcf, dstf)
    dinv = _tc_degsum(deg.reshape(NW, NH // 128, 128)).reshape(NH, 1)[:N]
    h = _tc_layer1(agg1, dinv, x, W1_l.T, W1_r.T, b1_l.reshape(1, D))
    agg2, = _AGG(h, srcf, dstf)
    out = _tc_layer2(agg2, dinv, h, W2_l.T, W2_r.T, b2_l.reshape(1, D))
    return out


# speedup vs baseline: 1.0276x; 1.0276x over previous
import functools

import jax
import jax.numpy as jnp
from jax import lax
from jax.experimental import pallas as pl
from jax.experimental.pallas import tpu as pltpu
from jax.experimental.pallas import tpu_sc as plsc

N = 10000
E = 320000
D = 128
NC = 2
NS = 16
NW = NC * NS
EPW = E // NW
BLK = 80
NBLK = EPW // BLK
GC = 5
IC = NBLK // GC
R0 = 632
NH = 10240
RLAST = N - (NS - 1) * R0


def _sc_aggregate(with_deg):
    mesh = plsc.VectorSubcoreMesh(core_axis_name="c", subcore_axis_name="s")

    out_type = [jax.ShapeDtypeStruct((NC, N, D), jnp.float32)]
    scratch = [
        pltpu.VMEM((IC, BLK), jnp.int32),
        pltpu.VMEM((IC, BLK), jnp.int32),
        pltpu.VMEM((BLK, D), jnp.float32),
        pltpu.VMEM((BLK, D), jnp.float32),
        pltpu.VMEM((BLK, D), jnp.float32),
        pltpu.VMEM_SHARED((N, D), jnp.float32),
        pltpu.SemaphoreType.DMA,
        pltpu.SemaphoreType.DMA,
        pltpu.SemaphoreType.DMA,
        pltpu.SemaphoreType.DMA,
        pltpu.SemaphoreType.DMA,
        pltpu.SemaphoreType.DMA,
    ]
    if with_deg:
        out_type.append(jax.ShapeDtypeStruct((NW, NH), jnp.float32))
        scratch += [
            pltpu.VMEM((NH,), jnp.float32),
        ]

    @functools.partial(
        pl.kernel, out_type=out_type, mesh=mesh, scratch_types=scratch,
        compiler_params=pltpu.CompilerParams(needs_layout_passes=False))
    def agg(x_hbm, edge_hbm, *rest):
        if with_deg:
            (out_hbm, deg_hbm, src_v, dst_v, rows0, rows1, rows2,
             acc_sh, gs0, gs1, gs2, ss0, ss1, ss2, hist_v) = rest
        else:
            (out_hbm, src_v, dst_v, rows0, rows1, rows2,
             acc_sh, gs0, gs1, gs2, ss0, ss1, ss2) = rest
        c = lax.axis_index("c")
        s = lax.axis_index("s")
        wid = c * NS + s
        zrows16 = jnp.zeros((16,), jnp.float32)

        def zrow_body(r, carry):
            for j in range(D // 16):
                rows0[r, pl.ds(j * 16, 16)] = zrows16
            return carry

        lax.fori_loop(0, BLK, zrow_body, 0)

        @pl.when(s < NS - 1)
        def _():
            for k in range(R0 // BLK):
                pltpu.sync_copy(rows0, acc_sh.at[pl.ds(s * R0 + k * BLK, BLK)])
            rem = R0 - (R0 // BLK) * BLK
            pltpu.sync_copy(rows0.at[pl.ds(0, rem)],
                            acc_sh.at[pl.ds(s * R0 + R0 - rem, rem)])

        @pl.when(s == NS - 1)
        def _():
            base = (NS - 1) * R0
            for k in range(RLAST // BLK):
                pltpu.sync_copy(rows0, acc_sh.at[pl.ds(base + k * BLK, BLK)])
            rem = RLAST - (RLAST // BLK) * BLK
            pltpu.sync_copy(rows0.at[pl.ds(0, rem)],
                            acc_sh.at[pl.ds(base + RLAST - rem, rem)])

        if with_deg:
            zeros16 = jnp.zeros((16,), jnp.float32)

            def zero_body(i, carry):
                hist_v[pl.ds(i * 16, 16)] = zeros16
                return carry

            lax.fori_loop(0, NH // 16, zero_body, 0)
            ones16 = jnp.full((16,), 1.0, jnp.float32)
        plsc.subcore_barrier()

        def deg_count(i):
            if with_deg:
                for j in range(BLK // 16):
                    chunk = dst_v[i, pl.ds(j * 16, 16)]
                    plsc.addupdate_scatter(hist_v, [chunk], ones16)

        rows = [rows0, rows1, rows2]
        gs = [gs0, gs1, gs2]
        ss = [ss0, ss1, ss2]

        def issue_gather(j, u):
            pltpu.async_copy(x_hbm.at[src_v.at[j]], rows[u], gs[u])

        def wait_gather(j, u):
            pltpu.make_async_copy(x_hbm.at[src_v.at[j]], rows[u], gs[u]).wait()

        def issue_scatter(j, u):
            pltpu.async_copy(rows[u], acc_sh.at[dst_v.at[j]], ss[u], add=True)

        def wait_scatter(j, u):
            pltpu.make_async_copy(rows[u], acc_sh.at[dst_v.at[j]], ss[u]).wait()

        for g in range(GC):
            pltpu.sync_copy(edge_hbm.at[0, wid, g], src_v)
            pltpu.sync_copy(edge_hbm.at[1, wid, g], dst_v)
            issue_gather(0, 0)
            issue_gather(1, 1)
            wait_gather(0, 0)
            issue_scatter(0, 0)
            deg_count(0)
            issue_gather(2, 2)
            wait_gather(1, 1)
            issue_scatter(1, 1)
            deg_count(1)
            wait_scatter(0, 0)
            issue_gather(3, 0)
            wait_gather(2, 2)
            issue_scatter(2, 2)
            deg_count(2)

            def body(k, carry):
                for u in range(3):
                    j = 3 * k + u
                    wait_scatter(j - 2, (u + 1) % 3)
                    issue_gather(j + 1, (u + 1) % 3)
                    wait_gather(j, u)
                    issue_scatter(j, u)
                    deg_count(j)
                return carry

            lax.fori_loop(1, IC // 3, body, 0)
            wait_scatter(22, 1)
            wait_gather(24, 0)
            issue_scatter(24, 0)
            deg_count(24)
            wait_scatter(23, 2)
            wait_scatter(24, 0)

        if with_deg:
            pltpu.sync_copy(hist_v, deg_hbm.at[wid])

        plsc.subcore_barrier()

        @pl.when(s < NS - 1)
        def _():
            pltpu.sync_copy(acc_sh.at[pl.ds(s * R0, R0)],
                            out_hbm.at[c, pl.ds(s * R0, R0)])

        @pl.when(s == NS - 1)
        def _():
            pltpu.sync_copy(acc_sh.at[pl.ds((NS - 1) * R0, RLAST)],
                            out_hbm.at[c, pl.ds((NS - 1) * R0, RLAST)])

    return agg


_AGG_DEG = _sc_aggregate(True)
_AGG = _sc_aggregate(False)

_R = 2000


def _tc_degsum(degr):
    def body(deg_ref, o_ref):
        dsum = jnp.sum(deg_ref[...], axis=0)
        o_ref[...] = 1.0 / jnp.maximum(dsum, 1.0)

    return pl.pallas_call(
        body,
        out_shape=jax.ShapeDtypeStruct((NH // 128, 128), jnp.float32),
    )(degr)


def _tc_layer1(aggp, dinv, x, w_l_t, w_r_t, b):
    def body(agg_ref, dinv_ref, x_ref, wl_ref, wr_ref, b_ref, h_ref):
        a = agg_ref[0] + agg_ref[1]
        mean = a * dinv_ref[...]
        h = (jnp.dot(mean, wl_ref[...], preferred_element_type=jnp.float32)
             + jnp.dot(x_ref[...], wr_ref[...], preferred_element_type=jnp.float32)
             + b_ref[...])
        h_ref[...] = jnp.maximum(h, 0.0)

    return pl.pallas_call(
        body,
        grid=(N // _R,),
        in_specs=[
            pl.BlockSpec((NC, _R, D), lambda i: (0, i, 0)),
            pl.BlockSpec((_R, 1), lambda i: (i, 0)),
            pl.BlockSpec((_R, D), lambda i: (i, 0)),
            pl.BlockSpec((D, D), lambda i: (0, 0)),
            pl.BlockSpec((D, D), lambda i: (0, 0)),
            pl.BlockSpec((1, D), lambda i: (0, 0)),
        ],
        out_specs=pl.BlockSpec((_R, D), lambda i: (i, 0)),
        out_shape=jax.ShapeDtypeStruct((N, D), jnp.float32),
    )(aggp, dinv, x, w_l_t, w_r_t, b)


def _tc_layer2(aggp, dinv, h, w_l_t, w_r_t, b):
    def body(agg_ref, dinv_ref, h_ref, wl_ref, wr_ref, b_ref, o_ref):
        a = agg_ref[0] + agg_ref[1]
        mean = a * dinv_ref[...]
        o_ref[...] = (jnp.dot(mean, wl_ref[...], preferred_element_type=jnp.float32)
                      + jnp.dot(h_ref[...], wr_ref[...], preferred_element_type=jnp.float32)
                      + b_ref[...])

    return pl.pallas_call(
        body,
        grid=(N // _R,),
        in_specs=[
            pl.BlockSpec((NC, _R, D), lambda i: (0, i, 0)),
            pl.BlockSpec((_R, 1), lambda i: (i, 0)),
            pl.BlockSpec((_R, D), lambda i: (i, 0)),
            pl.BlockSpec((D, D), lambda i: (0, 0)),
            pl.BlockSpec((D, D), lambda i: (0, 0)),
            pl.BlockSpec((1, D), lambda i: (0, 0)),
        ],
        out_specs=pl.BlockSpec((_R, D), lambda i: (i, 0)),
        out_shape=jax.ShapeDtypeStruct((N, D), jnp.float32),
    )(aggp, dinv, h, w_l_t, w_r_t, b)


def kernel(x, edge_index, W1_l, b1_l, W1_r, W2_l, b2_l, W2_r):
    er = edge_index.reshape(2, NW, GC, IC, BLK)

    agg1, deg = _AGG_DEG(x, er)
    dinv = _tc_degsum(deg.reshape(NW, NH // 128, 128)).reshape(NH, 1)[:N]
    h = _tc_layer1(agg1, dinv, x, W1_l.T, W1_r.T, b1_l.reshape(1, D))
    agg2, = _AGG(h, er)
    out = _tc_layer2(agg2, dinv, h, W2_l.T, W2_r.T, b2_l.reshape(1, D))
    return out


# speedup vs baseline: 1.0552x; 1.0268x over previous
import functools

import jax
import jax.numpy as jnp
from jax import lax
from jax.experimental import pallas as pl
from jax.experimental.pallas import tpu as pltpu
from jax.experimental.pallas import tpu_sc as plsc

N = 10000
E = 320000
D = 128
NC = 2
NS = 16
NW = NC * NS
EPW = E // NW
BLK = 80
NBLK = EPW // BLK
GC = 5
IC = NBLK // GC
R0 = 632
NH = 10240
RLAST = N - (NS - 1) * R0


def _sc_aggregate(with_deg):
    mesh = plsc.VectorSubcoreMesh(core_axis_name="c", subcore_axis_name="s")

    out_type = [jax.ShapeDtypeStruct((NC, N, D), jnp.float32)]
    scratch = [
        pltpu.VMEM((IC, BLK), jnp.int32),
        pltpu.VMEM((IC, BLK), jnp.int32),
        pltpu.VMEM((BLK, D), jnp.float32),
        pltpu.VMEM((BLK, D), jnp.float32),
        pltpu.VMEM((BLK, D), jnp.float32),
        pltpu.VMEM_SHARED((N, D), jnp.float32),
        pltpu.SemaphoreType.DMA,
        pltpu.SemaphoreType.DMA,
        pltpu.SemaphoreType.DMA,
        pltpu.SemaphoreType.DMA,
        pltpu.SemaphoreType.DMA,
        pltpu.SemaphoreType.DMA,
    ]
    if with_deg:
        out_type.append(jax.ShapeDtypeStruct((NW, NH // D, D), jnp.float32))
        scratch += [
            pltpu.VMEM((NH // D, D), jnp.float32),
        ]

    @functools.partial(
        pl.kernel, out_type=out_type, mesh=mesh, scratch_types=scratch,
        compiler_params=pltpu.CompilerParams(needs_layout_passes=False))
    def agg(x_hbm, edge_hbm, *rest):
        if with_deg:
            (out_hbm, deg_hbm, src_v, dst_v, rows0, rows1, rows2,
             acc_sh, gs0, gs1, gs2, ss0, ss1, ss2, hist_v) = rest
        else:
            (out_hbm, src_v, dst_v, rows0, rows1, rows2,
             acc_sh, gs0, gs1, gs2, ss0, ss1, ss2) = rest
        c = lax.axis_index("c")
        s = lax.axis_index("s")
        wid = c * NS + s
        rows = [rows0, rows1, rows2]
        gs = [gs0, gs1, gs2]
        ss = [ss0, ss1, ss2]

        def issue_gather(j, u):
            pltpu.async_copy(x_hbm.at[src_v.at[j]], rows[u], gs[u])

        def wait_gather(j, u):
            pltpu.make_async_copy(x_hbm.at[src_v.at[j]], rows[u], gs[u]).wait()

        def issue_scatter(j, u):
            pltpu.async_copy(rows[u], acc_sh.at[dst_v.at[j]], ss[u], add=True)

        def wait_scatter(j, u):
            pltpu.make_async_copy(rows[u], acc_sh.at[dst_v.at[j]], ss[u]).wait()

        pltpu.sync_copy(edge_hbm.at[0, wid, 0], src_v)
        pltpu.sync_copy(edge_hbm.at[1, wid, 0], dst_v)
        issue_gather(0, 1)
        issue_gather(1, 2)
        zrows16 = jnp.zeros((16,), jnp.float32)

        def zrow_body(r, carry):
            for j in range(D // 16):
                rows0[r, pl.ds(j * 16, 16)] = zrows16
            return carry

        lax.fori_loop(0, BLK, zrow_body, 0)

        @pl.when(s < NS - 1)
        def _():
            for k in range(R0 // BLK):
                pltpu.sync_copy(rows0, acc_sh.at[pl.ds(s * R0 + k * BLK, BLK)])
            rem = R0 - (R0 // BLK) * BLK
            pltpu.sync_copy(rows0.at[pl.ds(0, rem)],
                            acc_sh.at[pl.ds(s * R0 + R0 - rem, rem)])

        @pl.when(s == NS - 1)
        def _():
            base = (NS - 1) * R0
            for k in range(RLAST // BLK):
                pltpu.sync_copy(rows0, acc_sh.at[pl.ds(base + k * BLK, BLK)])
            rem = RLAST - (RLAST // BLK) * BLK
            pltpu.sync_copy(rows0.at[pl.ds(0, rem)],
                            acc_sh.at[pl.ds(base + RLAST - rem, rem)])

        if with_deg:
            zeros16 = jnp.zeros((16,), jnp.float32)

            def zero_body(i, carry):
                for j in range(D // 16):
                    hist_v[i, pl.ds(j * 16, 16)] = zeros16
                return carry

            lax.fori_loop(0, NH // D, zero_body, 0)
            ones16 = jnp.full((16,), 1.0, jnp.float32)

        def deg_count(i):
            if with_deg:
                for j in range(BLK // 16):
                    chunk = dst_v[i, pl.ds(j * 16, 16)]
                    plsc.addupdate_scatter(
                        hist_v, [chunk >> 7, chunk & (D - 1)], ones16)

        plsc.subcore_barrier()

        for g in range(GC):
            if g > 0:
                pltpu.sync_copy(edge_hbm.at[0, wid, g], src_v)
                pltpu.sync_copy(edge_hbm.at[1, wid, g], dst_v)
                issue_gather(0, 1)
                issue_gather(1, 2)
            wait_gather(0, 1)
            issue_scatter(0, 1)
            deg_count(0)
            issue_gather(2, 0)
            wait_gather(1, 2)
            issue_scatter(1, 2)
            deg_count(1)
            wait_scatter(0, 1)
            issue_gather(3, 1)
            wait_gather(2, 0)
            issue_scatter(2, 0)
            deg_count(2)

            def body(k, carry):
                for u in range(3):
                    j = 3 * k + u
                    wait_scatter(j - 2, (u + 2) % 3)
                    issue_gather(j + 1, (u + 2) % 3)
                    wait_gather(j, (u + 1) % 3)
                    issue_scatter(j, (u + 1) % 3)
                    deg_count(j)
                return carry

            lax.fori_loop(1, IC // 3, body, 0)
            wait_scatter(22, 2)
            wait_gather(24, 1)
            issue_scatter(24, 1)
            deg_count(24)
            wait_scatter(23, 0)
            wait_scatter(24, 1)

        if with_deg:
            pltpu.sync_copy(hist_v, deg_hbm.at[wid])

        plsc.subcore_barrier()

        @pl.when(s < NS - 1)
        def _():
            pltpu.sync_copy(acc_sh.at[pl.ds(s * R0, R0)],
                            out_hbm.at[c, pl.ds(s * R0, R0)])

        @pl.when(s == NS - 1)
        def _():
            pltpu.sync_copy(acc_sh.at[pl.ds((NS - 1) * R0, RLAST)],
                            out_hbm.at[c, pl.ds((NS - 1) * R0, RLAST)])

    return agg


_AGG_DEG = _sc_aggregate(True)
_AGG = _sc_aggregate(False)

_R = 2000


def _tc_degsum(degr):
    def body(deg_ref, o_ref):
        dsum = jnp.sum(deg_ref[...], axis=0)
        o_ref[...] = 1.0 / jnp.maximum(dsum, 1.0)

    return pl.pallas_call(
        body,
        out_shape=jax.ShapeDtypeStruct((NH // 128, 128), jnp.float32),
    )(degr)


def _tc_layer1(aggp, dinv, x, w_l_t, w_r_t, b):
    def body(agg_ref, dinv_ref, x_ref, wl_ref, wr_ref, b_ref, h_ref):
        a = agg_ref[0] + agg_ref[1]
        mean = a * dinv_ref[...]
        h = (jnp.dot(mean, wl_ref[...], preferred_element_type=jnp.float32)
             + jnp.dot(x_ref[...], wr_ref[...], preferred_element_type=jnp.float32)
             + b_ref[...])
        h_ref[...] = jnp.maximum(h, 0.0)

    return pl.pallas_call(
        body,
        grid=(N // _R,),
        in_specs=[
            pl.BlockSpec((NC, _R, D), lambda i: (0, i, 0)),
            pl.BlockSpec((_R, 1), lambda i: (i, 0)),
            pl.BlockSpec((_R, D), lambda i: (i, 0)),
            pl.BlockSpec((D, D), lambda i: (0, 0)),
            pl.BlockSpec((D, D), lambda i: (0, 0)),
            pl.BlockSpec((1, D), lambda i: (0, 0)),
        ],
        out_specs=pl.BlockSpec((_R, D), lambda i: (i, 0)),
        out_shape=jax.ShapeDtypeStruct((N, D), jnp.float32),
    )(aggp, dinv, x, w_l_t, w_r_t, b)


def _tc_layer2(aggp, dinv, h, w_l_t, w_r_t, b):
    def body(agg_ref, dinv_ref, h_ref, wl_ref, wr_ref, b_ref, o_ref):
        a = agg_ref[0] + agg_ref[1]
        mean = a * dinv_ref[...]
        o_ref[...] = (jnp.dot(mean, wl_ref[...], preferred_element_type=jnp.float32)
                      + jnp.dot(h_ref[...], wr_ref[...], preferred_element_type=jnp.float32)
                      + b_ref[...])

    return pl.pallas_call(
        body,
        grid=(N // _R,),
        in_specs=[
            pl.BlockSpec((NC, _R, D), lambda i: (0, i, 0)),
            pl.BlockSpec((_R, 1), lambda i: (i, 0)),
            pl.BlockSpec((_R, D), lambda i: (i, 0)),
            pl.BlockSpec((D, D), lambda i: (0, 0)),
            pl.BlockSpec((D, D), lambda i: (0, 0)),
            pl.BlockSpec((1, D), lambda i: (0, 0)),
        ],
        out_specs=pl.BlockSpec((_R, D), lambda i: (i, 0)),
        out_shape=jax.ShapeDtypeStruct((N, D), jnp.float32),
    )(aggp, dinv, h, w_l_t, w_r_t, b)


def kernel(x, edge_index, W1_l, b1_l, W1_r, W2_l, b2_l, W2_r):
    er = edge_index.reshape(2, NW, GC, IC, BLK)

    agg1, deg = _AGG_DEG(x, er)
    dinv = _tc_degsum(deg).reshape(NH, 1)
    h = _tc_layer1(agg1, dinv, x, W1_l.T, W1_r.T, b1_l.reshape(1, D))
    agg2, = _AGG(h, er)
    out = _tc_layer2(agg2, dinv, h, W2_l.T, W2_r.T, b2_l.reshape(1, D))
    return out


# speedup vs baseline: 1.0754x; 1.0191x over previous
import functools

import jax
import jax.numpy as jnp
from jax import lax
from jax.experimental import pallas as pl
from jax.experimental.pallas import tpu as pltpu
from jax.experimental.pallas import tpu_sc as plsc

N = 10000
E = 320000
D = 128
NC = 2
NS = 16
NW = NC * NS
EPW = E // NW
BLK = 80
NBLK = EPW // BLK
GC = 5
IC = NBLK // GC
R0 = 632
NH = 10240
RLAST = N - (NS - 1) * R0


def _sc_aggregate(with_deg):
    mesh = plsc.VectorSubcoreMesh(core_axis_name="c", subcore_axis_name="s")

    out_type = [jax.ShapeDtypeStruct((NC, N, D), jnp.float32)]
    nidx = 1 if with_deg else 2
    scratch = [
        pltpu.VMEM((IC, BLK), jnp.int32),
        pltpu.VMEM((IC, BLK), jnp.int32),
    ] * nidx + [
        pltpu.VMEM((BLK, D), jnp.float32),
        pltpu.VMEM((BLK, D), jnp.float32),
        pltpu.VMEM((BLK, D), jnp.float32),
        pltpu.VMEM_SHARED((N, D), jnp.float32),
        pltpu.SemaphoreType.DMA,
        pltpu.SemaphoreType.DMA,
        pltpu.SemaphoreType.DMA,
        pltpu.SemaphoreType.DMA,
        pltpu.SemaphoreType.DMA,
        pltpu.SemaphoreType.DMA,
        pltpu.SemaphoreType.DMA,
    ]
    if with_deg:
        out_type.append(jax.ShapeDtypeStruct((NW, NH // D, D), jnp.float32))
        scratch += [
            pltpu.VMEM((NH // D, D), jnp.float32),
        ]

    @functools.partial(
        pl.kernel, out_type=out_type, mesh=mesh, scratch_types=scratch,
        compiler_params=pltpu.CompilerParams(needs_layout_passes=False))
    def agg(x_hbm, edge_hbm, *rest):
        if with_deg:
            (out_hbm, deg_hbm, src_a, dst_a, rows0, rows1,
             rows2, acc_sh, gs0, gs1, gs2, ss0, ss1, ss2, isem, hist_v) = rest
            src_b, dst_b = src_a, dst_a
        else:
            (out_hbm, src_a, dst_a, src_b, dst_b, rows0, rows1,
             rows2, acc_sh, gs0, gs1, gs2, ss0, ss1, ss2, isem) = rest
        c = lax.axis_index("c")
        s = lax.axis_index("s")
        wid = c * NS + s
        rows = [rows0, rows1, rows2]
        gs = [gs0, gs1, gs2]
        ss = [ss0, ss1, ss2]

        def issue_gather(sv, j, u):
            pltpu.async_copy(x_hbm.at[sv.at[j]], rows[u], gs[u])

        def wait_gather(sv, j, u):
            pltpu.make_async_copy(x_hbm.at[sv.at[j]], rows[u], gs[u]).wait()

        def issue_scatter(dv, j, u):
            pltpu.async_copy(rows[u], acc_sh.at[dv.at[j]], ss[u], add=True)

        def wait_scatter(dv, j, u):
            pltpu.make_async_copy(rows[u], acc_sh.at[dv.at[j]], ss[u]).wait()

        pairs = [(src_a, dst_a), (src_b, dst_b)]
        pltpu.sync_copy(edge_hbm.at[0, wid, 0], src_a)
        pltpu.sync_copy(edge_hbm.at[1, wid, 0], dst_a)
        issue_gather(src_a, 0, 1)
        issue_gather(src_a, 1, 2)
        if not with_deg:
            pltpu.async_copy(edge_hbm.at[0, wid, 1], src_b, isem)
            pltpu.async_copy(edge_hbm.at[1, wid, 1], dst_b, isem)
        zrows16 = jnp.zeros((16,), jnp.float32)

        def zrow_body(r, carry):
            for j in range(D // 16):
                rows0[r, pl.ds(j * 16, 16)] = zrows16
            return carry

        lax.fori_loop(0, BLK, zrow_body, 0)

        @pl.when(s < NS - 1)
        def _():
            for k in range(R0 // BLK):
                pltpu.sync_copy(rows0, acc_sh.at[pl.ds(s * R0 + k * BLK, BLK)])
            rem = R0 - (R0 // BLK) * BLK
            pltpu.sync_copy(rows0.at[pl.ds(0, rem)],
                            acc_sh.at[pl.ds(s * R0 + R0 - rem, rem)])

        @pl.when(s == NS - 1)
        def _():
            base = (NS - 1) * R0
            for k in range(RLAST // BLK):
                pltpu.sync_copy(rows0, acc_sh.at[pl.ds(base + k * BLK, BLK)])
            rem = RLAST - (RLAST // BLK) * BLK
            pltpu.sync_copy(rows0.at[pl.ds(0, rem)],
                            acc_sh.at[pl.ds(base + RLAST - rem, rem)])

        if with_deg:
            zeros16 = jnp.zeros((16,), jnp.float32)

            def zero_body(i, carry):
                for j in range(D // 16):
                    hist_v[i, pl.ds(j * 16, 16)] = zeros16
                return carry

            lax.fori_loop(0, NH // D, zero_body, 0)
            ones16 = jnp.full((16,), 1.0, jnp.float32)

        def deg_count(dv, i):
            if with_deg:
                for j in range(BLK // 16):
                    chunk = dv[i, pl.ds(j * 16, 16)]
                    plsc.addupdate_scatter(
                        hist_v, [chunk >> 7, chunk & (D - 1)], ones16)

        plsc.subcore_barrier()

        for g in range(GC):
            sv, dv = pairs[g % 2]
            if g > 0 and with_deg:
                pltpu.sync_copy(edge_hbm.at[0, wid, g], sv)
                pltpu.sync_copy(edge_hbm.at[1, wid, g], dv)
                issue_gather(sv, 0, 1)
                issue_gather(sv, 1, 2)
            elif g > 0:
                pltpu.make_async_copy(edge_hbm.at[0, wid, g], sv, isem).wait()
                pltpu.make_async_copy(edge_hbm.at[1, wid, g], dv, isem).wait()
                issue_gather(sv, 0, 1)
                issue_gather(sv, 1, 2)
            if g + 1 < GC and not with_deg:
                nsv, ndv = pairs[(g + 1) % 2]
                if g > 0:
                    pltpu.async_copy(edge_hbm.at[0, wid, g + 1], nsv, isem)
                    pltpu.async_copy(edge_hbm.at[1, wid, g + 1], ndv, isem)
            wait_gather(sv, 0, 1)
            issue_scatter(dv, 0, 1)
            deg_count(dv, 0)
            issue_gather(sv, 2, 0)
            wait_gather(sv, 1, 2)
            issue_scatter(dv, 1, 2)
            deg_count(dv, 1)
            wait_scatter(dv, 0, 1)
            issue_gather(sv, 3, 1)
            wait_gather(sv, 2, 0)
            issue_scatter(dv, 2, 0)
            deg_count(dv, 2)

            def body(k, carry):
                for u in range(3):
                    j = 3 * k + u
                    wait_scatter(dv, j - 2, (u + 2) % 3)
                    issue_gather(sv, j + 1, (u + 2) % 3)
                    wait_gather(sv, j, (u + 1) % 3)
                    issue_scatter(dv, j, (u + 1) % 3)
                    deg_count(dv, j)
                return carry

            lax.fori_loop(1, IC // 3, body, 0)
            wait_scatter(dv, 22, 2)
            wait_gather(sv, 24, 1)
            issue_scatter(dv, 24, 1)
            deg_count(dv, 24)
            wait_scatter(dv, 23, 0)
            wait_scatter(dv, 24, 1)

        if with_deg:
            pltpu.sync_copy(hist_v, deg_hbm.at[wid])

        plsc.subcore_barrier()

        @pl.when(s < NS - 1)
        def _():
            pltpu.sync_copy(acc_sh.at[pl.ds(s * R0, R0)],
                            out_hbm.at[c, pl.ds(s * R0, R0)])

        @pl.when(s == NS - 1)
        def _():
            pltpu.sync_copy(acc_sh.at[pl.ds((NS - 1) * R0, RLAST)],
                            out_hbm.at[c, pl.ds((NS - 1) * R0, RLAST)])

    return agg


_AGG_DEG = _sc_aggregate(True)
_AGG = _sc_aggregate(False)

_R = 2000


def _tc_degsum(degr):
    def body(deg_ref, o_ref):
        dsum = jnp.sum(deg_ref[...], axis=0)
        o_ref[...] = 1.0 / jnp.maximum(dsum, 1.0)

    return pl.pallas_call(
        body,
        out_shape=jax.ShapeDtypeStruct((NH // 128, 128), jnp.float32),
    )(degr)


def _tc_layer1(aggp, dinv, x, w_l_t, w_r_t, b):
    def body(agg_ref, dinv_ref, x_ref, wl_ref, wr_ref, b_ref, h_ref):
        a = agg_ref[0] + agg_ref[1]
        mean = a * dinv_ref[...]
        h = (jnp.dot(mean, wl_ref[...], preferred_element_type=jnp.float32)
             + jnp.dot(x_ref[...], wr_ref[...], preferred_element_type=jnp.float32)
             + b_ref[...])
        h_ref[...] = jnp.maximum(h, 0.0)

    return pl.pallas_call(
        body,
        grid=(N // _R,),
        in_specs=[
            pl.BlockSpec((NC, _R, D), lambda i: (0, i, 0)),
            pl.BlockSpec((_R, 1), lambda i: (i, 0)),
            pl.BlockSpec((_R, D), lambda i: (i, 0)),
            pl.BlockSpec((D, D), lambda i: (0, 0)),
            pl.BlockSpec((D, D), lambda i: (0, 0)),
            pl.BlockSpec((1, D), lambda i: (0, 0)),
        ],
        out_specs=pl.BlockSpec((_R, D), lambda i: (i, 0)),
        out_shape=jax.ShapeDtypeStruct((N, D), jnp.float32),
    )(aggp, dinv, x, w_l_t, w_r_t, b)


def _tc_layer2(aggp, dinv, h, w_l_t, w_r_t, b):
    def body(agg_ref, dinv_ref, h_ref, wl_ref, wr_ref, b_ref, o_ref):
        a = agg_ref[0] + agg_ref[1]
        mean = a * dinv_ref[...]
        o_ref[...] = (jnp.dot(mean, wl_ref[...], preferred_element_type=jnp.float32)
                      + jnp.dot(h_ref[...], wr_ref[...], preferred_element_type=jnp.float32)
                      + b_ref[...])

    return pl.pallas_call(
        body,
        grid=(N // _R,),
        in_specs=[
            pl.BlockSpec((NC, _R, D), lambda i: (0, i, 0)),
            pl.BlockSpec((_R, 1), lambda i: (i, 0)),
            pl.BlockSpec((_R, D), lambda i: (i, 0)),
            pl.BlockSpec((D, D), lambda i: (0, 0)),
            pl.BlockSpec((D, D), lambda i: (0, 0)),
            pl.BlockSpec((1, D), lambda i: (0, 0)),
        ],
        out_specs=pl.BlockSpec((_R, D), lambda i: (i, 0)),
        out_shape=jax.ShapeDtypeStruct((N, D), jnp.float32),
    )(aggp, dinv, h, w_l_t, w_r_t, b)


def kernel(x, edge_index, W1_l, b1_l, W1_r, W2_l, b2_l, W2_r):
    er = edge_index.reshape(2, NW, GC, IC, BLK)

    agg1, deg = _AGG_DEG(x, er)
    dinv = _tc_degsum(deg).reshape(NH, 1)
    h = _tc_layer1(agg1, dinv, x, W1_l.T, W1_r.T, b1_l.reshape(1, D))
    agg2, = _AGG(h, er)
    out = _tc_layer2(agg2, dinv, h, W2_l.T, W2_r.T, b2_l.reshape(1, D))
    return out
